# Initial kernel scaffold; baseline (speedup 1.0000x reference)
import jax
import jax.numpy as jnp
from jax.experimental import pallas as pl


def kernel(x, edge_index, W1, b1, W2, b2, W3, b3, Wout, bout):
    raise NotImplementedError("write your pallas kernel here")


# speedup vs baseline: 15.1848x; 15.1848x over previous
import functools

import jax
import jax.numpy as jnp
from jax import lax
from jax.experimental import pallas as pl
from jax.experimental.pallas import tpu as pltpu
from jax.experimental.pallas import tpu_sc as plsc

N_NODES = 10000
N_EDGES = 320000
NP = 10240
CHUNK = 128
NWORK = 32
CPW = 79
E_PAD = NWORK * CPW * CHUNK
RPT = NP // 16
NB = 1024


def _sc_mesh():
    return plsc.VectorSubcoreMesh(core_axis_name="c", subcore_axis_name="s")


def _zero_rows(buf, nrows, ncols):
    zvec = jnp.zeros((16,), jnp.float32)

    def body(i, carry):
        for j in range(ncols // 16):
            buf[i, pl.ds(j * 16, 16)] = zvec
        return carry

    lax.fori_loop(0, nrows, body, 0)


def _make_deg():
    @functools.partial(
        pl.kernel,
        out_type=(
            jax.ShapeDtypeStruct((NP, 16), jnp.float32),
            jax.ShapeDtypeStruct((NP, 16), jnp.float32),
        ),
        mesh=_sc_mesh(),
        scratch_types=[
            pltpu.VMEM((CHUNK,), jnp.int32),
            pltpu.VMEM((CHUNK, 16), jnp.float32),
            pltpu.VMEM((RPT, 16), jnp.float32),
            pltpu.VMEM_SHARED((NP, 16), jnp.float32),
        ],
        compiler_params=pltpu.CompilerParams(use_tc_tiling_on_sc=False),
    )
    def deg_kernel(dst_hbm, out_a, out_b, dst_v, ones_v, zbuf, acc_sh):
        c = lax.axis_index("c")
        s = lax.axis_index("s")
        w = s * 2 + c

        e0 = jnp.where(lax.iota(jnp.int32, 16) == 0, 1.0, 0.0).astype(jnp.float32)

        def fill(i, carry):
            ones_v[i, pl.ds(0, 16)] = e0
            return carry

        lax.fori_loop(0, CHUNK, fill, 0)
        _zero_rows(zbuf, RPT, 16)
        pltpu.sync_copy(zbuf, acc_sh.at[pl.ds(s * RPT, RPT)])
        plsc.subcore_barrier()

        def body(t, carry):
            base = (w * CPW + t) * CHUNK
            pltpu.sync_copy(dst_hbm.at[pl.ds(base, CHUNK)], dst_v)
            pltpu.sync_copy(ones_v, acc_sh.at[dst_v], add=True)
            return carry

        lax.fori_loop(0, CPW, body, 0)
        plsc.subcore_barrier()

        @pl.when(c == 0)
        def _():
            pltpu.sync_copy(acc_sh.at[pl.ds(s * RPT, RPT)], out_a.at[pl.ds(s * RPT, RPT)])

        @pl.when(c == 1)
        def _():
            pltpu.sync_copy(acc_sh.at[pl.ds(s * RPT, RPT)], out_b.at[pl.ds(s * RPT, RPT)])

    return deg_kernel


def _make_agg(d):
    @functools.partial(
        pl.kernel,
        out_type=(
            jax.ShapeDtypeStruct((NP, d), jnp.float32),
            jax.ShapeDtypeStruct((NP, d), jnp.float32),
        ),
        mesh=_sc_mesh(),
        scratch_types=[
            pltpu.VMEM((CHUNK,), jnp.int32),
            pltpu.VMEM((CHUNK,), jnp.int32),
            pltpu.VMEM((CHUNK, d), jnp.float32),
            pltpu.VMEM((RPT, d), jnp.float32),
            pltpu.VMEM_SHARED((NP, d), jnp.float32),
            pltpu.SemaphoreType.DMA,
        ],
        compiler_params=pltpu.CompilerParams(use_tc_tiling_on_sc=False),
    )
    def agg_kernel(z_hbm, src_hbm, dst_hbm, out_a, out_b,
                   src_v, dst_v, rows_v, zbuf, acc_sh, sem):
        c = lax.axis_index("c")
        s = lax.axis_index("s")
        w = s * 2 + c

        _zero_rows(zbuf, RPT, d)
        pltpu.sync_copy(zbuf, acc_sh.at[pl.ds(s * RPT, RPT)])
        plsc.subcore_barrier()

        def body(t, carry):
            base = (w * CPW + t) * CHUNK
            pltpu.sync_copy(src_hbm.at[pl.ds(base, CHUNK)], src_v)
            pltpu.sync_copy(dst_hbm.at[pl.ds(base, CHUNK)], dst_v)
            pltpu.async_copy(z_hbm.at[src_v], rows_v, sem).wait()
            pltpu.sync_copy(rows_v, acc_sh.at[dst_v], add=True)
            return carry

        lax.fori_loop(0, CPW, body, 0)
        plsc.subcore_barrier()

        @pl.when(c == 0)
        def _():
            pltpu.sync_copy(acc_sh.at[pl.ds(s * RPT, RPT)], out_a.at[pl.ds(s * RPT, RPT)])

        @pl.when(c == 1)
        def _():
            pltpu.sync_copy(acc_sh.at[pl.ds(s * RPT, RPT)], out_b.at[pl.ds(s * RPT, RPT)])

    return agg_kernel


def _dis_from_deg(dga, dgb):
    deg = jnp.sum(dga + dgb, axis=1, keepdims=True) + 1.0
    return lax.rsqrt(deg)


def _tc_first_body(x_ref, w_ref, dga_ref, dgb_ref, o_ref):
    dis = _dis_from_deg(dga_ref[...], dgb_ref[...])
    o_ref[...] = jnp.dot(x_ref[...], w_ref[...],
                         preferred_element_type=jnp.float32) * dis


def _tc_mid_body(aa_ref, ab_ref, z_ref, dga_ref, dgb_ref, b_ref, w_ref, o_ref):
    dis = _dis_from_deg(dga_ref[...], dgb_ref[...])
    h = jnp.maximum((aa_ref[...] + ab_ref[...] + z_ref[...]) * dis + b_ref[...], 0.0)
    o_ref[...] = jnp.dot(h, w_ref[...], preferred_element_type=jnp.float32) * dis


def _tc_last_body(aa_ref, ab_ref, z_ref, dga_ref, dgb_ref, b_ref, w_ref,
                  bo_ref, o_ref):
    dis = _dis_from_deg(dga_ref[...], dgb_ref[...])
    h = jnp.maximum((aa_ref[...] + ab_ref[...] + z_ref[...]) * dis + b_ref[...], 0.0)
    o_ref[...] = jnp.dot(h, w_ref[...], preferred_element_type=jnp.float32) + bo_ref[...]


def _row_spec(dcols):
    return pl.BlockSpec((NB, dcols), lambda i: (i, 0))


def _full_spec(r, ccols):
    return pl.BlockSpec((r, ccols), lambda i: (0, 0))


def _tc_first(x_pad, w1, dga, dgb):
    return pl.pallas_call(
        _tc_first_body,
        grid=(NP // NB,),
        in_specs=[_row_spec(128), _full_spec(128, 64), _row_spec(16), _row_spec(16)],
        out_specs=_row_spec(64),
        out_shape=jax.ShapeDtypeStruct((NP, 64), jnp.float32),
    )(x_pad, w1, dga, dgb)


def _tc_mid(aa, ab, z, dga, dgb, b, w, din, dout):
    return pl.pallas_call(
        _tc_mid_body,
        grid=(NP // NB,),
        in_specs=[_row_spec(din), _row_spec(din), _row_spec(din),
                  _row_spec(16), _row_spec(16),
                  _full_spec(1, din), _full_spec(din, dout)],
        out_specs=_row_spec(dout),
        out_shape=jax.ShapeDtypeStruct((NP, dout), jnp.float32),
    )(aa, ab, z, dga, dgb, b.reshape(1, din), w)


def _tc_last(aa, ab, z, dga, dgb, b, w, bo):
    return pl.pallas_call(
        _tc_last_body,
        grid=(NP // NB,),
        in_specs=[_row_spec(16), _row_spec(16), _row_spec(16),
                  _row_spec(16), _row_spec(16),
                  _full_spec(1, 16), _full_spec(16, 1), _full_spec(1, 1)],
        out_specs=_row_spec(1),
        out_shape=jax.ShapeDtypeStruct((NP, 1), jnp.float32),
    )(aa, ab, z, dga, dgb, b.reshape(1, 16), w, bo.reshape(1, 1))


_deg_call = _make_deg()
_agg64 = _make_agg(64)
_agg32 = _make_agg(32)
_agg16 = _make_agg(16)


def kernel(x, edge_index, W1, b1, W2, b2, W3, b3, Wout, bout):
    src = edge_index[0].astype(jnp.int32)
    dst = edge_index[1].astype(jnp.int32)
    pad = E_PAD - N_EDGES
    sink = jnp.full((pad,), N_NODES, jnp.int32)
    src = jnp.concatenate([src, sink])
    dst = jnp.concatenate([dst, sink])
    x_pad = jnp.pad(x, ((0, NP - N_NODES), (0, 0)))

    dga, dgb = _deg_call(dst)
    z1 = _tc_first(x_pad, W1, dga, dgb)
    a1a, a1b = _agg64(z1, src, dst)
    z2 = _tc_mid(a1a, a1b, z1, dga, dgb, b1, W2, 64, 32)
    a2a, a2b = _agg32(z2, src, dst)
    z3 = _tc_mid(a2a, a2b, z2, dga, dgb, b2, W3, 32, 16)
    a3a, a3b = _agg16(z3, src, dst)
    out = _tc_last(a3a, a3b, z3, dga, dgb, b3, Wout, bout)
    return out[:N_NODES]
